# baseline (speedup 1.0000x reference)
import functools

import jax
import jax.numpy as jnp
from jax import lax
from jax.experimental import pallas as pl
from jax.experimental.pallas import tpu as pltpu
from jax.experimental.pallas import tpu_sc as plsc

N = 100000
E = 1600000
NC = 2
NS = 16
BN = 2048
N_PAD = 100352
ROWS_T = N_PAD // NS
W = 4
BPC = 12
P = BPC // 2
NCH = 17
CR = BPC * W
RT = NCH * CR
E_T = RT * 128
E_PAD = NS * E_T
IDX_BLK = 8
DNB = RT // IDX_BLK
H = 32
HH = 16


def _mesh():
    return plsc.VectorSubcoreMesh(
        core_axis_name="c", subcore_axis_name="s", num_cores=NC, num_subcores=NS
    )


def _sc_agg(u_lo, u_hi, src2d, dst2d, zeros2d):
    @functools.partial(
        pl.kernel,
        out_type=(
            jax.ShapeDtypeStruct((N_PAD, HH), jnp.float32),
            jax.ShapeDtypeStruct((N_PAD, HH), jnp.float32),
        ),
        mesh=_mesh(),
        compiler_params=pltpu.CompilerParams(use_tc_tiling_on_sc=False),
        scratch_types=[
            pltpu.VMEM((CR, 128), jnp.int32),
            pltpu.VMEM((CR, 128), jnp.int32),
            pltpu.VMEM((W, 128, HH), jnp.float32),
            pltpu.VMEM((W, 128, HH), jnp.float32),
            pltpu.VMEM_SHARED((N_PAD, HH), jnp.float32),
            pltpu.SemaphoreType.DMA,
            pltpu.SemaphoreType.DMA,
            pltpu.SemaphoreType.DMA,
            pltpu.SemaphoreType.DMA,
        ],
    )
    def k(u_lo_h, u_hi_h, src_h, dst_h, z_h, out_lo, out_hi,
          srcC, dstC, rowsA, rowsB, acc,
          gsA, gsB, asA, asB):
        c = lax.axis_index("c")
        s = lax.axis_index("s")
        base = s * ROWS_T
        pltpu.sync_copy(z_h.at[pl.ds(0, ROWS_T)], acc.at[pl.ds(base, ROWS_T)])
        plsc.subcore_barrier()

        def edge_pass(table, out):
            tile_r0 = s * RT

            def fire_g(blk, rows, sem):
                for j in range(W):
                    pltpu.async_copy(table.at[srcC.at[blk * W + j]],
                                     rows.at[j], sem)

            def wait_g(rows, sem):
                for j in range(W):
                    pltpu.make_async_copy(table.at[srcC.at[0]],
                                          rows.at[j], sem).wait()

            def fire_a(blk, rows, sem):
                for j in range(W):
                    pltpu.async_copy(rows.at[j],
                                     acc.at[dstC.at[blk * W + j]], sem,
                                     add=True)

            def wait_a(rows, sem):
                for j in range(W):
                    pltpu.make_async_copy(rows.at[j], acc.at[dstC.at[0]],
                                          sem).wait()

            def chunk(ci, _):
                r0 = tile_r0 + ci * CR
                pltpu.sync_copy(src_h.at[pl.ds(r0, CR)], srcC)
                pltpu.sync_copy(dst_h.at[pl.ds(r0, CR)], dstC)
                fire_g(0, rowsA, gsA)

                def pair(p, _):
                    blk0 = 2 * p
                    wait_g(rowsA, gsA)

                    @pl.when(p > 0)
                    def _():
                        wait_a(rowsB, asB)

                    fire_g(blk0 + 1, rowsB, gsB)
                    fire_a(blk0, rowsA, asA)
                    wait_g(rowsB, gsB)
                    wait_a(rowsA, asA)

                    @pl.when(p < P - 1)
                    def _():
                        fire_g(blk0 + 2, rowsA, gsA)

                    fire_a(blk0 + 1, rowsB, asB)
                    return 0

                lax.fori_loop(0, P, pair, 0)
                wait_a(rowsB, asB)
                return 0

            lax.fori_loop(0, NCH, chunk, 0)
            plsc.subcore_barrier()
            pltpu.sync_copy(acc.at[pl.ds(base, ROWS_T)],
                            out.at[pl.ds(base, ROWS_T)])

        @pl.when(c == 0)
        def _():
            edge_pass(u_lo_h, out_lo)

        @pl.when(c == 1)
        def _():
            edge_pass(u_hi_h, out_hi)

    return k(u_lo, u_hi, src2d, dst2d, zeros2d)


def _sc_degrees(src2d, dst2d, zeros1d):
    @functools.partial(
        pl.kernel,
        out_type=(
            jax.ShapeDtypeStruct((N_PAD,), jnp.float32),
            jax.ShapeDtypeStruct((N_PAD,), jnp.float32),
        ),
        mesh=_mesh(),
        compiler_params=pltpu.CompilerParams(use_tc_tiling_on_sc=False),
        scratch_types=[
            pltpu.VMEM((IDX_BLK, 128), jnp.int32),
            pltpu.VMEM((128,), jnp.float32),
            pltpu.VMEM_SHARED((N_PAD,), jnp.float32),
            pltpu.SemaphoreType.DMA,
        ],
    )
    def k(src_h, dst_h, z_h, out_do, out_di, idx_v, ones_v, acc, asem):
        c = lax.axis_index("c")
        s = lax.axis_index("s")
        base = s * ROWS_T
        for kk in range(8):
            ones_v[pl.ds(kk * 16, 16)] = jnp.ones((16,), jnp.float32)
        pltpu.sync_copy(z_h.at[pl.ds(0, ROWS_T)], acc.at[pl.ds(base, ROWS_T)])
        plsc.subcore_barrier()

        def count_pass(idx_h, out):
            def blk(i, _):
                r0 = s * RT + i * IDX_BLK
                pltpu.sync_copy(idx_h.at[pl.ds(r0, IDX_BLK)], idx_v)
                a = [
                    pltpu.async_copy(ones_v, acc.at[idx_v.at[j]], asem,
                                     add=True)
                    for j in range(IDX_BLK)
                ]
                for d in a:
                    d.wait()
                return 0

            lax.fori_loop(0, DNB, blk, 0)
            plsc.subcore_barrier()
            pltpu.sync_copy(acc.at[pl.ds(base, ROWS_T)],
                            out.at[pl.ds(base, ROWS_T)])

        @pl.when(c == 0)
        def _():
            count_pass(src_h, out_do)

        @pl.when(c == 1)
        def _():
            count_pass(dst_h, out_di)

    return k(src2d, dst2d, zeros1d)


def _tc_norms(deg_out, deg_in):
    def body(do_r, di_r, ns_r, nd_r):
        ns_r[...] = lax.rsqrt(jnp.maximum(do_r[...], 1.0))
        nd_r[...] = lax.rsqrt(jnp.maximum(di_r[...], 1.0))

    shp = jax.ShapeDtypeStruct((N_PAD // 128, 128), jnp.float32)
    return pl.pallas_call(body, out_shape=(shp, shp))(
        deg_out.reshape(N_PAD // 128, 128), deg_in.reshape(N_PAD // 128, 128)
    )


def _tc_first(x_pad, w1p, norm_src):
    def body(x_r, w_r, ns_r, ulo_r, uhi_r):
        t = jnp.dot(x_r[...], w_r[...], preferred_element_type=jnp.float32)
        u = t * ns_r[...]
        ulo_r[...] = u[:, :HH]
        uhi_r[...] = u[:, HH:]

    grid = (N_PAD // BN,)
    shp = jax.ShapeDtypeStruct((N_PAD, HH), jnp.float32)
    return pl.pallas_call(
        body,
        grid=grid,
        in_specs=[
            pl.BlockSpec((BN, 64), lambda i: (i, 0)),
            pl.BlockSpec((64, H), lambda i: (0, 0)),
            pl.BlockSpec((BN, 1), lambda i: (i, 0)),
        ],
        out_specs=(
            pl.BlockSpec((BN, HH), lambda i: (i, 0)),
            pl.BlockSpec((BN, HH), lambda i: (i, 0)),
        ),
        out_shape=(shp, shp),
    )(x_pad, w1p, norm_src)


def _tc_mid(agg_lo, agg_hi, norm_dst, norm_src, w, b):
    def body(alo_r, ahi_r, nd_r, ns_r, w_r, b_r, ulo_r, uhi_r):
        agg = jnp.concatenate([alo_r[...], ahi_r[...]], axis=1)
        h = jax.nn.relu(agg * nd_r[...] + b_r[...])
        u = jnp.dot(h, w_r[...], preferred_element_type=jnp.float32) * ns_r[...]
        ulo_r[...] = u[:, :HH]
        uhi_r[...] = u[:, HH:]

    grid = (N_PAD // BN,)
    shp = jax.ShapeDtypeStruct((N_PAD, HH), jnp.float32)
    return pl.pallas_call(
        body,
        grid=grid,
        in_specs=[
            pl.BlockSpec((BN, HH), lambda i: (i, 0)),
            pl.BlockSpec((BN, HH), lambda i: (i, 0)),
            pl.BlockSpec((BN, 1), lambda i: (i, 0)),
            pl.BlockSpec((BN, 1), lambda i: (i, 0)),
            pl.BlockSpec((H, H), lambda i: (0, 0)),
            pl.BlockSpec((1, H), lambda i: (0, 0)),
        ],
        out_specs=(
            pl.BlockSpec((BN, HH), lambda i: (i, 0)),
            pl.BlockSpec((BN, HH), lambda i: (i, 0)),
        ),
        out_shape=(shp, shp),
    )(agg_lo, agg_hi, norm_dst, norm_src, w, b)


def _tc_last(agg_lo, agg_hi, norm_dst, b, fcw_p, fcb_p):
    def body(alo_r, ahi_r, nd_r, b_r, w_r, fb_r, o_r):
        agg = jnp.concatenate([alo_r[...], ahi_r[...]], axis=1)
        h = jax.nn.relu(agg * nd_r[...] + b_r[...])
        o_r[...] = jnp.dot(h, w_r[...], preferred_element_type=jnp.float32) \
            + fb_r[...]

    grid = (N_PAD // BN,)
    return pl.pallas_call(
        body,
        grid=grid,
        in_specs=[
            pl.BlockSpec((BN, HH), lambda i: (i, 0)),
            pl.BlockSpec((BN, HH), lambda i: (i, 0)),
            pl.BlockSpec((BN, 1), lambda i: (i, 0)),
            pl.BlockSpec((1, H), lambda i: (0, 0)),
            pl.BlockSpec((H, 8), lambda i: (0, 0)),
            pl.BlockSpec((1, 8), lambda i: (0, 0)),
        ],
        out_specs=pl.BlockSpec((BN, 8), lambda i: (i, 0)),
        out_shape=jax.ShapeDtypeStruct((N_PAD, 8), jnp.float32),
    )(agg_lo, agg_hi, norm_dst, b, fcw_p, fcb_p)


@jax.jit
def kernel(x, edge_index, W1, Wr, bs, fcW, fcb):
    src = edge_index[0].astype(jnp.int32)
    dst = edge_index[1].astype(jnp.int32)
    pad = jnp.full((E_PAD - E,), N_PAD - 1, jnp.int32)
    src2d = jnp.concatenate([src, pad]).reshape(E_PAD // 128, 128)
    dst2d = jnp.concatenate([dst, pad]).reshape(E_PAD // 128, 128)
    zeros2d = jnp.zeros((ROWS_T, HH), jnp.float32)
    zeros1d = jnp.zeros((ROWS_T,), jnp.float32)

    deg_out, deg_in = _sc_degrees(src2d, dst2d, zeros1d)
    ns2d, nd2d = _tc_norms(deg_out, deg_in)
    norm_src = ns2d.reshape(N_PAD, 1)
    norm_dst = nd2d.reshape(N_PAD, 1)

    x_pad = jnp.pad(x, ((0, N_PAD - N), (0, 64 - x.shape[1])))
    w1p = jnp.pad(W1, ((0, 64 - W1.shape[0]), (0, 0)))
    u_lo, u_hi = _tc_first(x_pad, w1p, norm_src)

    for l in range(9):
        agg_lo, agg_hi = _sc_agg(u_lo, u_hi, src2d, dst2d, zeros2d)
        u_lo, u_hi = _tc_mid(agg_lo, agg_hi, norm_dst, norm_src,
                             Wr[l], bs[l][None, :])

    agg_lo, agg_hi = _sc_agg(u_lo, u_hi, src2d, dst2d, zeros2d)
    fcw_p = jnp.pad(fcW, ((0, 0), (0, 8 - fcW.shape[1])))
    fcb_p = jnp.pad(fcb, ((0, 8 - fcb.shape[0],)))[None, :]
    out = _tc_last(agg_lo, agg_hi, norm_dst, bs[9][None, :], fcw_p, fcb_p)
    return out[:N, :2]


# speedup vs baseline: 1.4075x; 1.4075x over previous
import functools

import jax
import jax.numpy as jnp
from jax import lax
from jax.experimental import pallas as pl
from jax.experimental.pallas import tpu as pltpu
from jax.experimental.pallas import tpu_sc as plsc

N = 100000
E = 1600000
NC = 2
NS = 16
BN = 2048
N_PAD = 100352
ROWS_T = N_PAD // NS
W = 12
NB = 66
RT = NB * W
E_T = RT * 128
E_PAD = NS * E_T
IDX_BLK = 8
DNB = RT // IDX_BLK
H = 32
HH = 16


def _mesh():
    return plsc.VectorSubcoreMesh(
        core_axis_name="c", subcore_axis_name="s", num_cores=NC, num_subcores=NS
    )


def _sc_agg(u_lo, u_hi, src2d, dst2d, zeros2d):
    @functools.partial(
        pl.kernel,
        out_type=(
            jax.ShapeDtypeStruct((N_PAD, HH), jnp.float32),
            jax.ShapeDtypeStruct((N_PAD, HH), jnp.float32),
        ),
        mesh=_mesh(),
        compiler_params=pltpu.CompilerParams(use_tc_tiling_on_sc=False),
        scratch_types=[
            pltpu.VMEM((W, 128), jnp.int32),
            pltpu.VMEM((W, 128), jnp.int32),
            pltpu.VMEM((W, 128, HH), jnp.float32),
            pltpu.VMEM_SHARED((N_PAD, HH), jnp.float32),
            pltpu.SemaphoreType.DMA,
            pltpu.SemaphoreType.DMA,
        ],
    )
    def k(u_lo_h, u_hi_h, src_h, dst_h, z_h, out_lo, out_hi,
          src_v, dst_v, rows, acc, gsem, asem):
        c = lax.axis_index("c")
        s = lax.axis_index("s")
        base = s * ROWS_T
        pltpu.sync_copy(z_h.at[pl.ds(0, ROWS_T)], acc.at[pl.ds(base, ROWS_T)])
        plsc.subcore_barrier()

        def edge_pass(table, out):
            def blk(i, _):
                r0 = s * RT + i * W
                pltpu.sync_copy(src_h.at[pl.ds(r0, W)], src_v)
                pltpu.sync_copy(dst_h.at[pl.ds(r0, W)], dst_v)
                g = [
                    pltpu.async_copy(table.at[src_v.at[j]], rows.at[j], gsem)
                    for j in range(W)
                ]
                for d in g:
                    d.wait()
                a = [
                    pltpu.async_copy(rows.at[j], acc.at[dst_v.at[j]], asem,
                                     add=True)
                    for j in range(W)
                ]
                for d in a:
                    d.wait()
                return 0

            lax.fori_loop(0, NB, blk, 0)
            plsc.subcore_barrier()
            pltpu.sync_copy(acc.at[pl.ds(base, ROWS_T)],
                            out.at[pl.ds(base, ROWS_T)])

        @pl.when(c == 0)
        def _():
            edge_pass(u_lo_h, out_lo)

        @pl.when(c == 1)
        def _():
            edge_pass(u_hi_h, out_hi)

    return k(u_lo, u_hi, src2d, dst2d, zeros2d)


def _sc_degrees(src2d, dst2d, zeros1d):
    @functools.partial(
        pl.kernel,
        out_type=(
            jax.ShapeDtypeStruct((N_PAD,), jnp.float32),
            jax.ShapeDtypeStruct((N_PAD,), jnp.float32),
        ),
        mesh=_mesh(),
        compiler_params=pltpu.CompilerParams(use_tc_tiling_on_sc=False),
        scratch_types=[
            pltpu.VMEM((IDX_BLK, 128), jnp.int32),
            pltpu.VMEM((128,), jnp.float32),
            pltpu.VMEM_SHARED((N_PAD,), jnp.float32),
            pltpu.SemaphoreType.DMA,
        ],
    )
    def k(src_h, dst_h, z_h, out_do, out_di, idx_v, ones_v, acc, asem):
        c = lax.axis_index("c")
        s = lax.axis_index("s")
        base = s * ROWS_T
        for kk in range(8):
            ones_v[pl.ds(kk * 16, 16)] = jnp.ones((16,), jnp.float32)
        pltpu.sync_copy(z_h.at[pl.ds(0, ROWS_T)], acc.at[pl.ds(base, ROWS_T)])
        plsc.subcore_barrier()

        def count_pass(idx_h, out):
            def blk(i, _):
                r0 = s * RT + i * IDX_BLK
                pltpu.sync_copy(idx_h.at[pl.ds(r0, IDX_BLK)], idx_v)
                a = [
                    pltpu.async_copy(ones_v, acc.at[idx_v.at[j]], asem,
                                     add=True)
                    for j in range(IDX_BLK)
                ]
                for d in a:
                    d.wait()
                return 0

            lax.fori_loop(0, DNB, blk, 0)
            plsc.subcore_barrier()
            pltpu.sync_copy(acc.at[pl.ds(base, ROWS_T)],
                            out.at[pl.ds(base, ROWS_T)])

        @pl.when(c == 0)
        def _():
            count_pass(src_h, out_do)

        @pl.when(c == 1)
        def _():
            count_pass(dst_h, out_di)

    return k(src2d, dst2d, zeros1d)


def _tc_norms(deg_out, deg_in):
    def body(do_r, di_r, ns_r, nd_r):
        ns_r[...] = lax.rsqrt(jnp.maximum(do_r[...], 1.0))
        nd_r[...] = lax.rsqrt(jnp.maximum(di_r[...], 1.0))

    shp = jax.ShapeDtypeStruct((N_PAD // 128, 128), jnp.float32)
    return pl.pallas_call(body, out_shape=(shp, shp))(
        deg_out.reshape(N_PAD // 128, 128), deg_in.reshape(N_PAD // 128, 128)
    )


def _tc_first(x_pad, w1p, norm_src):
    def body(x_r, w_r, ns_r, ulo_r, uhi_r):
        t = jnp.dot(x_r[...], w_r[...], preferred_element_type=jnp.float32)
        u = t * ns_r[...]
        ulo_r[...] = u[:, :HH]
        uhi_r[...] = u[:, HH:]

    grid = (N_PAD // BN,)
    shp = jax.ShapeDtypeStruct((N_PAD, HH), jnp.float32)
    return pl.pallas_call(
        body,
        grid=grid,
        in_specs=[
            pl.BlockSpec((BN, 64), lambda i: (i, 0)),
            pl.BlockSpec((64, H), lambda i: (0, 0)),
            pl.BlockSpec((BN, 1), lambda i: (i, 0)),
        ],
        out_specs=(
            pl.BlockSpec((BN, HH), lambda i: (i, 0)),
            pl.BlockSpec((BN, HH), lambda i: (i, 0)),
        ),
        out_shape=(shp, shp),
    )(x_pad, w1p, norm_src)


def _tc_mid(agg_lo, agg_hi, norm_dst, norm_src, w, b):
    def body(alo_r, ahi_r, nd_r, ns_r, w_r, b_r, ulo_r, uhi_r):
        agg = jnp.concatenate([alo_r[...], ahi_r[...]], axis=1)
        h = jax.nn.relu(agg * nd_r[...] + b_r[...])
        u = jnp.dot(h, w_r[...], preferred_element_type=jnp.float32) * ns_r[...]
        ulo_r[...] = u[:, :HH]
        uhi_r[...] = u[:, HH:]

    grid = (N_PAD // BN,)
    shp = jax.ShapeDtypeStruct((N_PAD, HH), jnp.float32)
    return pl.pallas_call(
        body,
        grid=grid,
        in_specs=[
            pl.BlockSpec((BN, HH), lambda i: (i, 0)),
            pl.BlockSpec((BN, HH), lambda i: (i, 0)),
            pl.BlockSpec((BN, 1), lambda i: (i, 0)),
            pl.BlockSpec((BN, 1), lambda i: (i, 0)),
            pl.BlockSpec((H, H), lambda i: (0, 0)),
            pl.BlockSpec((1, H), lambda i: (0, 0)),
        ],
        out_specs=(
            pl.BlockSpec((BN, HH), lambda i: (i, 0)),
            pl.BlockSpec((BN, HH), lambda i: (i, 0)),
        ),
        out_shape=(shp, shp),
    )(agg_lo, agg_hi, norm_dst, norm_src, w, b)


def _tc_last(agg_lo, agg_hi, norm_dst, b, fcw_p, fcb_p):
    def body(alo_r, ahi_r, nd_r, b_r, w_r, fb_r, o_r):
        agg = jnp.concatenate([alo_r[...], ahi_r[...]], axis=1)
        h = jax.nn.relu(agg * nd_r[...] + b_r[...])
        o_r[...] = jnp.dot(h, w_r[...], preferred_element_type=jnp.float32) \
            + fb_r[...]

    grid = (N_PAD // BN,)
    return pl.pallas_call(
        body,
        grid=grid,
        in_specs=[
            pl.BlockSpec((BN, HH), lambda i: (i, 0)),
            pl.BlockSpec((BN, HH), lambda i: (i, 0)),
            pl.BlockSpec((BN, 1), lambda i: (i, 0)),
            pl.BlockSpec((1, H), lambda i: (0, 0)),
            pl.BlockSpec((H, 8), lambda i: (0, 0)),
            pl.BlockSpec((1, 8), lambda i: (0, 0)),
        ],
        out_specs=pl.BlockSpec((BN, 8), lambda i: (i, 0)),
        out_shape=jax.ShapeDtypeStruct((N_PAD, 8), jnp.float32),
    )(agg_lo, agg_hi, norm_dst, b, fcw_p, fcb_p)


@jax.jit
def kernel(x, edge_index, W1, Wr, bs, fcW, fcb):
    src = edge_index[0].astype(jnp.int32)
    dst = edge_index[1].astype(jnp.int32)
    pad = jnp.full((E_PAD - E,), N_PAD - 1, jnp.int32)
    src2d = jnp.concatenate([src, pad]).reshape(E_PAD // 128, 128)
    dst2d = jnp.concatenate([dst, pad]).reshape(E_PAD // 128, 128)
    zeros2d = jnp.zeros((ROWS_T, HH), jnp.float32)
    zeros1d = jnp.zeros((ROWS_T,), jnp.float32)

    deg_out, deg_in = _sc_degrees(src2d, dst2d, zeros1d)
    ns2d, nd2d = _tc_norms(deg_out, deg_in)
    norm_src = ns2d.reshape(N_PAD, 1)
    norm_dst = nd2d.reshape(N_PAD, 1)

    x_pad = jnp.pad(x, ((0, N_PAD - N), (0, 64 - x.shape[1])))
    w1p = jnp.pad(W1, ((0, 64 - W1.shape[0]), (0, 0)))
    u_lo, u_hi = _tc_first(x_pad, w1p, norm_src)

    for l in range(9):
        agg_lo, agg_hi = _sc_agg(u_lo, u_hi, src2d, dst2d, zeros2d)
        u_lo, u_hi = _tc_mid(agg_lo, agg_hi, norm_dst, norm_src,
                             Wr[l], bs[l][None, :])

    agg_lo, agg_hi = _sc_agg(u_lo, u_hi, src2d, dst2d, zeros2d)
    fcw_p = jnp.pad(fcW, ((0, 0), (0, 8 - fcW.shape[1])))
    fcb_p = jnp.pad(fcb, ((0, 8 - fcb.shape[0],)))[None, :]
    out = _tc_last(agg_lo, agg_hi, norm_dst, bs[9][None, :], fcw_p, fcb_p)
    return out[:N, :2]


# speedup vs baseline: 1.5709x; 1.1161x over previous
import functools

import jax
import jax.numpy as jnp
from jax import lax
from jax.experimental import pallas as pl
from jax.experimental.pallas import tpu as pltpu
from jax.experimental.pallas import tpu_sc as plsc

N = 100000
E = 1600000
NC = 2
NS = 16
BN = 2048
N_PAD = 100352
ROWS_T = N_PAD // NS
W = 8
HW2 = W // 2
NB = 98
RT = NB * W
E_T = RT * 128
E_PAD = NS * E_T
IDX_BLK = 8
DNB = RT // IDX_BLK
H = 32
HH = 16


def _mesh():
    return plsc.VectorSubcoreMesh(
        core_axis_name="c", subcore_axis_name="s", num_cores=NC, num_subcores=NS
    )


def _sc_agg(u_lo, u_hi, src2d, dst2d, zeros2d):
    @functools.partial(
        pl.kernel,
        out_type=(
            jax.ShapeDtypeStruct((N_PAD, HH), jnp.float32),
            jax.ShapeDtypeStruct((N_PAD, HH), jnp.float32),
        ),
        mesh=_mesh(),
        compiler_params=pltpu.CompilerParams(use_tc_tiling_on_sc=False),
        scratch_types=[
            pltpu.VMEM((W, 128), jnp.int32),
            pltpu.VMEM((W, 128), jnp.int32),
            pltpu.VMEM((W, 128, HH), jnp.float32),
            pltpu.VMEM_SHARED((N_PAD, HH), jnp.float32),
            pltpu.SemaphoreType.DMA,
            pltpu.SemaphoreType.DMA,
        ],
    )
    def k(u_lo_h, u_hi_h, src_h, dst_h, z_h, out_lo, out_hi,
          src_v, dst_v, rows, acc, gsem, asem):
        c = lax.axis_index("c")
        s = lax.axis_index("s")
        base = s * ROWS_T
        pltpu.sync_copy(z_h.at[pl.ds(0, ROWS_T)], acc.at[pl.ds(base, ROWS_T)])
        plsc.subcore_barrier()

        def edge_pass(table, out):
            def blk(i, _):
                r0 = s * RT + i * W
                pltpu.sync_copy(src_h.at[pl.ds(r0, W)], src_v)
                pltpu.sync_copy(dst_h.at[pl.ds(r0, W)], dst_v)
                g = [
                    pltpu.async_copy(table.at[src_v.at[j]], rows.at[j], gsem)
                    for j in range(W)
                ]
                for d in g:
                    d.wait()
                a = [
                    pltpu.async_copy(rows.at[j], acc.at[dst_v.at[j]], asem,
                                     add=True)
                    for j in range(W)
                ]
                for d in a:
                    d.wait()
                return 0

            lax.fori_loop(0, NB, blk, 0)
            plsc.subcore_barrier()
            pltpu.sync_copy(acc.at[pl.ds(base, ROWS_T)],
                            out.at[pl.ds(base, ROWS_T)])

        @pl.when(c == 0)
        def _():
            edge_pass(u_lo_h, out_lo)

        @pl.when(c == 1)
        def _():
            edge_pass(u_hi_h, out_hi)

    return k(u_lo, u_hi, src2d, dst2d, zeros2d)


def _sc_degrees(src2d, dst2d, zeros1d):
    @functools.partial(
        pl.kernel,
        out_type=(
            jax.ShapeDtypeStruct((N_PAD,), jnp.float32),
            jax.ShapeDtypeStruct((N_PAD,), jnp.float32),
        ),
        mesh=_mesh(),
        compiler_params=pltpu.CompilerParams(use_tc_tiling_on_sc=False),
        scratch_types=[
            pltpu.VMEM((IDX_BLK, 128), jnp.int32),
            pltpu.VMEM((128,), jnp.float32),
            pltpu.VMEM_SHARED((N_PAD,), jnp.float32),
            pltpu.SemaphoreType.DMA,
        ],
    )
    def k(src_h, dst_h, z_h, out_do, out_di, idx_v, ones_v, acc, asem):
        c = lax.axis_index("c")
        s = lax.axis_index("s")
        base = s * ROWS_T
        for kk in range(8):
            ones_v[pl.ds(kk * 16, 16)] = jnp.ones((16,), jnp.float32)
        pltpu.sync_copy(z_h.at[pl.ds(0, ROWS_T)], acc.at[pl.ds(base, ROWS_T)])
        plsc.subcore_barrier()

        def count_pass(idx_h, out):
            def blk(i, _):
                r0 = s * RT + i * IDX_BLK
                pltpu.sync_copy(idx_h.at[pl.ds(r0, IDX_BLK)], idx_v)
                a = [
                    pltpu.async_copy(ones_v, acc.at[idx_v.at[j]], asem,
                                     add=True)
                    for j in range(IDX_BLK)
                ]
                for d in a:
                    d.wait()
                return 0

            lax.fori_loop(0, DNB, blk, 0)
            plsc.subcore_barrier()
            pltpu.sync_copy(acc.at[pl.ds(base, ROWS_T)],
                            out.at[pl.ds(base, ROWS_T)])

        @pl.when(c == 0)
        def _():
            count_pass(src_h, out_do)

        @pl.when(c == 1)
        def _():
            count_pass(dst_h, out_di)

    return k(src2d, dst2d, zeros1d)


def _tc_norms(deg_out, deg_in):
    def body(do_r, di_r, ns_r, nd_r):
        ns_r[...] = lax.rsqrt(jnp.maximum(do_r[...], 1.0))
        nd_r[...] = lax.rsqrt(jnp.maximum(di_r[...], 1.0))

    shp = jax.ShapeDtypeStruct((N_PAD // 128, 128), jnp.float32)
    return pl.pallas_call(body, out_shape=(shp, shp))(
        deg_out.reshape(N_PAD // 128, 128), deg_in.reshape(N_PAD // 128, 128)
    )


def _tc_first(x_pad, w1p, norm_src):
    def body(x_r, w_r, ns_r, ulo_r, uhi_r):
        t = jnp.dot(x_r[...], w_r[...], preferred_element_type=jnp.float32)
        u = t * ns_r[...]
        ulo_r[...] = u[:, :HH]
        uhi_r[...] = u[:, HH:]

    grid = (N_PAD // BN,)
    shp = jax.ShapeDtypeStruct((N_PAD, HH), jnp.float32)
    return pl.pallas_call(
        body,
        grid=grid,
        in_specs=[
            pl.BlockSpec((BN, 64), lambda i: (i, 0)),
            pl.BlockSpec((64, H), lambda i: (0, 0)),
            pl.BlockSpec((BN, 1), lambda i: (i, 0)),
        ],
        out_specs=(
            pl.BlockSpec((BN, HH), lambda i: (i, 0)),
            pl.BlockSpec((BN, HH), lambda i: (i, 0)),
        ),
        out_shape=(shp, shp),
    )(x_pad, w1p, norm_src)


def _tc_mid(agg_lo, agg_hi, norm_dst, norm_src, w, b):
    def body(alo_r, ahi_r, nd_r, ns_r, w_r, b_r, ulo_r, uhi_r):
        agg = jnp.concatenate([alo_r[...], ahi_r[...]], axis=1)
        h = jax.nn.relu(agg * nd_r[...] + b_r[...])
        u = jnp.dot(h, w_r[...], preferred_element_type=jnp.float32) * ns_r[...]
        ulo_r[...] = u[:, :HH]
        uhi_r[...] = u[:, HH:]

    grid = (N_PAD // BN,)
    shp = jax.ShapeDtypeStruct((N_PAD, HH), jnp.float32)
    return pl.pallas_call(
        body,
        grid=grid,
        in_specs=[
            pl.BlockSpec((BN, HH), lambda i: (i, 0)),
            pl.BlockSpec((BN, HH), lambda i: (i, 0)),
            pl.BlockSpec((BN, 1), lambda i: (i, 0)),
            pl.BlockSpec((BN, 1), lambda i: (i, 0)),
            pl.BlockSpec((H, H), lambda i: (0, 0)),
            pl.BlockSpec((1, H), lambda i: (0, 0)),
        ],
        out_specs=(
            pl.BlockSpec((BN, HH), lambda i: (i, 0)),
            pl.BlockSpec((BN, HH), lambda i: (i, 0)),
        ),
        out_shape=(shp, shp),
    )(agg_lo, agg_hi, norm_dst, norm_src, w, b)


def _tc_last(agg_lo, agg_hi, norm_dst, b, fcw_p, fcb_p):
    def body(alo_r, ahi_r, nd_r, b_r, w_r, fb_r, o_r):
        agg = jnp.concatenate([alo_r[...], ahi_r[...]], axis=1)
        h = jax.nn.relu(agg * nd_r[...] + b_r[...])
        o_r[...] = jnp.dot(h, w_r[...], preferred_element_type=jnp.float32) \
            + fb_r[...]

    grid = (N_PAD // BN,)
    return pl.pallas_call(
        body,
        grid=grid,
        in_specs=[
            pl.BlockSpec((BN, HH), lambda i: (i, 0)),
            pl.BlockSpec((BN, HH), lambda i: (i, 0)),
            pl.BlockSpec((BN, 1), lambda i: (i, 0)),
            pl.BlockSpec((1, H), lambda i: (0, 0)),
            pl.BlockSpec((H, 8), lambda i: (0, 0)),
            pl.BlockSpec((1, 8), lambda i: (0, 0)),
        ],
        out_specs=pl.BlockSpec((BN, 8), lambda i: (i, 0)),
        out_shape=jax.ShapeDtypeStruct((N_PAD, 8), jnp.float32),
    )(agg_lo, agg_hi, norm_dst, b, fcw_p, fcb_p)


@jax.jit
def kernel(x, edge_index, W1, Wr, bs, fcW, fcb):
    src = edge_index[0].astype(jnp.int32)
    dst = edge_index[1].astype(jnp.int32)
    pad = jnp.full((E_PAD - E,), N_PAD - 1, jnp.int32)
    src2d = jnp.concatenate([src, pad]).reshape(E_PAD // 128, 128)
    dst2d = jnp.concatenate([dst, pad]).reshape(E_PAD // 128, 128)
    zeros2d = jnp.zeros((ROWS_T, HH), jnp.float32)
    zeros1d = jnp.zeros((ROWS_T,), jnp.float32)

    deg_out, deg_in = _sc_degrees(src2d, dst2d, zeros1d)
    ns2d, nd2d = _tc_norms(deg_out, deg_in)
    norm_src = ns2d.reshape(N_PAD, 1)
    norm_dst = nd2d.reshape(N_PAD, 1)

    x_pad = jnp.pad(x, ((0, N_PAD - N), (0, 64 - x.shape[1])))
    w1p = jnp.pad(W1, ((0, 64 - W1.shape[0]), (0, 0)))
    u_lo, u_hi = _tc_first(x_pad, w1p, norm_src)

    for l in range(9):
        agg_lo, agg_hi = _sc_agg(u_lo, u_hi, src2d, dst2d, zeros2d)
        u_lo, u_hi = _tc_mid(agg_lo, agg_hi, norm_dst, norm_src,
                             Wr[l], bs[l][None, :])

    agg_lo, agg_hi = _sc_agg(u_lo, u_hi, src2d, dst2d, zeros2d)
    fcw_p = jnp.pad(fcW, ((0, 0), (0, 8 - fcW.shape[1])))
    fcb_p = jnp.pad(fcb, ((0, 8 - fcb.shape[0],)))[None, :]
    out = _tc_last(agg_lo, agg_hi, norm_dst, bs[9][None, :], fcw_p, fcb_p)
    return out[:N, :2]
